# baseline (speedup 1.0000x reference)
import dataclasses
import functools

import jax
import jax.numpy as jnp
from jax import lax
from jax.experimental import pallas as pl
from jax.experimental.pallas import tpu as pltpu
from jax.experimental.pallas import tpu_sc as plsc

K = 128
B = 16384
NC = 2
NS = 16
NW = NC * NS
ROWS_W = B // NW
CH = 32
NCH = ROWS_W // CH
GRP = 16
NGRP = CH // GRP
NBUF = 3
BCH = 128

_mesh = plsc.VectorSubcoreMesh(core_axis_name="c", subcore_axis_name="s")

_cp = pltpu.CompilerParams()
if "needs_layout_passes" in pltpu.CompilerParams.__dataclass_fields__:
    _cp = dataclasses.replace(_cp, needs_layout_passes=False)


@functools.partial(
    pl.kernel,
    out_type=jax.ShapeDtypeStruct((B,), jnp.float32),
    mesh=_mesh,
    compiler_params=_cp,
    scratch_types=[
        pltpu.VMEM((ROWS_W,), jnp.int32),
        pltpu.VMEM((ROWS_W,), jnp.int32),
        pltpu.VMEM((NBUF, CH, K), jnp.float32),
        pltpu.VMEM((NBUF, CH, K), jnp.float32),
        pltpu.VMEM((ROWS_W, K), jnp.float32),
        pltpu.VMEM((ROWS_W,), jnp.float32),
        pltpu.VMEM((NBUF, CH), jnp.float32),
        pltpu.VMEM((GRP * GRP,), jnp.float32),
        pltpu.SemaphoreType.DMA((NBUF,)),
        pltpu.SemaphoreType.DMA((NBUF,)),
        pltpu.SemaphoreType.DMA,
        pltpu.SemaphoreType.DMA,
    ],
)
def _irf_kernel(u_hbm, i_hbm, kp_hbm, th_hbm, a_hbm, b_hbm, out_hbm,
                uix, iix, th_v, a_v, kp_v, b_v, o_v, tr_v,
                isem, osem, xsem, bsem):
    wid = lax.axis_index("c") * NS + lax.axis_index("s")
    base_w = wid * ROWS_W
    cp_u = pltpu.async_copy(u_hbm.at[pl.ds(base_w, ROWS_W)], uix, xsem)
    cp_i = pltpu.async_copy(i_hbm.at[pl.ds(base_w, ROWS_W)], iix, xsem)
    cp_u.wait()
    cp_i.wait()
    lanes = lax.iota(jnp.int32, 16)

    cp_kp = pltpu.async_copy(kp_hbm.at[pl.ds(base_w, ROWS_W)], kp_v, bsem)
    b_cps = [
        pltpu.async_copy(b_hbm.at[iix.at[pl.ds(j * BCH, BCH)]],
                         b_v.at[pl.ds(j * BCH, BCH)], bsem)
        for j in range(ROWS_W // BCH)
    ]

    def issue(c, p):
        pltpu.async_copy(th_hbm.at[uix.at[pl.ds(c * CH, CH)]],
                         th_v.at[p], isem.at[p])
        pltpu.async_copy(a_hbm.at[iix.at[pl.ds(c * CH, CH)]],
                         a_v.at[p], isem.at[p])

    def wait_gathers(p):
        pltpu.make_async_copy(th_hbm.at[pl.ds(0, CH)], th_v.at[p],
                              isem.at[p]).wait()
        pltpu.make_async_copy(a_hbm.at[pl.ds(0, CH)], a_v.at[p],
                              isem.at[p]).wait()

    issue(0, 0)
    issue(1, 1)
    cp_kp.wait()
    for cp in b_cps:
        cp.wait()

    @pl.loop(0, NCH)
    def _(c):
        p = lax.rem(c, NBUF)

        @pl.when(c + 2 < NCH)
        def _():
            issue(c + 2, lax.rem(c + 2, NBUF))

        wait_gathers(p)

        @pl.when(c >= NBUF)
        def _():
            pltpu.make_async_copy(o_v.at[p], out_hbm.at[pl.ds(0, CH)],
                                  osem.at[p]).wait()

        @pl.loop(0, NGRP)
        def _(g):
            IL = 4
            for r in range(0, GRP, IL):
                rows = tuple(g * GRP + r + j for j in range(IL))
                acc = [jnp.zeros((GRP,), jnp.float32) for _ in range(IL)]
                for k in range(K // GRP):
                    for j in range(IL):
                        t = th_v[p, rows[j], pl.ds(k * GRP, GRP)]
                        av = a_v[p, rows[j], pl.ds(k * GRP, GRP)]
                        kv = kp_v[c * CH + rows[j], pl.ds(k * GRP, GRP)]
                        acc[j] = acc[j] + t * av * kv
                for j in range(IL):
                    plsc.store_scatter(tr_v, [lanes * GRP + (r + j)], acc[j])
            terms = [tr_v[pl.ds(i * GRP, GRP)] for i in range(GRP)]
            while len(terms) > 1:
                terms = [terms[i] + terms[i + 1]
                         for i in range(0, len(terms), 2)]
            s = terms[0]
            bb = b_v[pl.ds(c * CH + g * GRP, GRP)]
            o_v[p, pl.ds(g * GRP, GRP)] = 1.0 / (1.0 + jnp.exp(bb - s))

        pltpu.async_copy(o_v.at[p], out_hbm.at[pl.ds(base_w + c * CH, CH)],
                         osem.at[p])

    for p in range(NBUF):
        pltpu.make_async_copy(o_v.at[p], out_hbm.at[pl.ds(0, CH)],
                              osem.at[p]).wait()


@jax.jit
def kernel(user, item, input_knowledge_point, theta_w, a_w, b_w):
    return _irf_kernel(user.astype(jnp.int32), item.astype(jnp.int32),
                       input_knowledge_point, theta_w, a_w, b_w.reshape(-1))


# speedup vs baseline: 1.0402x; 1.0402x over previous
import dataclasses
import functools

import jax
import jax.numpy as jnp
from jax import lax
from jax.experimental import pallas as pl
from jax.experimental.pallas import tpu as pltpu
from jax.experimental.pallas import tpu_sc as plsc

K = 128
B = 16384
NC = 2
NS = 16
NW = NC * NS
ROWS_W = B // NW
CH = 32
NCH = ROWS_W // CH
GRP = 16
NGRP = CH // GRP
NBUF = 3

_mesh = plsc.VectorSubcoreMesh(core_axis_name="c", subcore_axis_name="s")

_cp = pltpu.CompilerParams()
if "needs_layout_passes" in pltpu.CompilerParams.__dataclass_fields__:
    _cp = dataclasses.replace(_cp, needs_layout_passes=False)


@functools.partial(
    pl.kernel,
    out_type=jax.ShapeDtypeStruct((B,), jnp.float32),
    mesh=_mesh,
    compiler_params=_cp,
    scratch_types=[
        pltpu.VMEM((ROWS_W,), jnp.int32),
        pltpu.VMEM((ROWS_W,), jnp.int32),
        pltpu.VMEM((NBUF, CH, K), jnp.float32),
        pltpu.VMEM((NBUF, CH, K), jnp.float32),
        pltpu.VMEM((NBUF, CH, K), jnp.float32),
        pltpu.VMEM((NBUF, CH), jnp.float32),
        pltpu.VMEM((NBUF, CH), jnp.float32),
        pltpu.VMEM((GRP * GRP,), jnp.float32),
        pltpu.SemaphoreType.DMA((NBUF,)),
        pltpu.SemaphoreType.DMA((NBUF,)),
        pltpu.SemaphoreType.DMA,
    ],
)
def _irf_kernel(u_hbm, i_hbm, kp_hbm, th_hbm, a_hbm, b_hbm, out_hbm,
                uix, iix, th_v, a_v, kp_v, b_v, o_v, tr_v, isem, osem, xsem):
    wid = lax.axis_index("c") * NS + lax.axis_index("s")
    base_w = wid * ROWS_W
    cp_u = pltpu.async_copy(u_hbm.at[pl.ds(base_w, ROWS_W)], uix, xsem)
    cp_i = pltpu.async_copy(i_hbm.at[pl.ds(base_w, ROWS_W)], iix, xsem)
    cp_u.wait()
    cp_i.wait()
    lanes = lax.iota(jnp.int32, 16)

    def issue(c, p):
        pltpu.async_copy(th_hbm.at[uix.at[pl.ds(c * CH, CH)]],
                         th_v.at[p], isem.at[p])
        pltpu.async_copy(a_hbm.at[iix.at[pl.ds(c * CH, CH)]],
                         a_v.at[p], isem.at[p])
        pltpu.async_copy(b_hbm.at[iix.at[pl.ds(c * CH, CH)]],
                         b_v.at[p], isem.at[p])
        pltpu.async_copy(kp_hbm.at[pl.ds(base_w + c * CH, CH)],
                         kp_v.at[p], isem.at[p])

    def wait_gathers(p):
        pltpu.make_async_copy(th_hbm.at[pl.ds(0, CH)], th_v.at[p],
                              isem.at[p]).wait()
        pltpu.make_async_copy(a_hbm.at[pl.ds(0, CH)], a_v.at[p],
                              isem.at[p]).wait()
        pltpu.make_async_copy(b_hbm.at[pl.ds(0, CH)], b_v.at[p],
                              isem.at[p]).wait()
        pltpu.make_async_copy(kp_hbm.at[pl.ds(0, CH)], kp_v.at[p],
                              isem.at[p]).wait()

    issue(0, 0)
    issue(1, 1)

    @pl.loop(0, NCH)
    def _(c):
        p = lax.rem(c, NBUF)

        @pl.when(c + 2 < NCH)
        def _():
            issue(c + 2, lax.rem(c + 2, NBUF))

        wait_gathers(p)

        @pl.when(c >= NBUF)
        def _():
            pltpu.make_async_copy(o_v.at[p], out_hbm.at[pl.ds(0, CH)],
                                  osem.at[p]).wait()

        @pl.loop(0, NGRP)
        def _(g):
            IL = 4
            for r in range(0, GRP, IL):
                rows = tuple(g * GRP + r + j for j in range(IL))
                acc = [jnp.zeros((GRP,), jnp.float32) for _ in range(IL)]
                for k in range(K // GRP):
                    for j in range(IL):
                        t = th_v[p, rows[j], pl.ds(k * GRP, GRP)]
                        av = a_v[p, rows[j], pl.ds(k * GRP, GRP)]
                        kv = kp_v[p, rows[j], pl.ds(k * GRP, GRP)]
                        acc[j] = acc[j] + t * av * kv
                for j in range(IL):
                    plsc.store_scatter(tr_v, [lanes * GRP + (r + j)], acc[j])
            terms = [tr_v[pl.ds(i * GRP, GRP)] for i in range(GRP)]
            while len(terms) > 1:
                terms = [terms[i] + terms[i + 1]
                         for i in range(0, len(terms), 2)]
            s = terms[0]
            bb = b_v[p, pl.ds(g * GRP, GRP)]
            o_v[p, pl.ds(g * GRP, GRP)] = 1.0 / (1.0 + jnp.exp(bb - s))

        pltpu.async_copy(o_v.at[p], out_hbm.at[pl.ds(base_w + c * CH, CH)],
                         osem.at[p])

    for p in range(NBUF):
        pltpu.make_async_copy(o_v.at[p], out_hbm.at[pl.ds(0, CH)],
                              osem.at[p]).wait()


@jax.jit
def kernel(user, item, input_knowledge_point, theta_w, a_w, b_w):
    return _irf_kernel(user.astype(jnp.int32), item.astype(jnp.int32),
                       input_knowledge_point, theta_w, a_w, b_w.reshape(-1))
